# baseline (speedup 1.0000x reference)
import functools

import jax
import jax.numpy as jnp
from jax import lax
from jax.experimental import pallas as pl
from jax.experimental.pallas import tpu as pltpu
from jax.experimental.pallas import tpu_sc as plsc

N_NODES = 100000
D = 128
G = 512
NCLS = 10
L = 16
NC = 2
NS = 16
NW = NC * NS
CB = 400
KMAX = 8
PER_W = CB * KMAX
NV = D // L


def _sc_segment_sum(x1d, seg):
    mesh = plsc.VectorSubcoreMesh(core_axis_name="c", subcore_axis_name="s")

    @functools.partial(
        pl.kernel,
        out_type=(
            jax.ShapeDtypeStruct((NC, G * D), jnp.float32),
            jax.ShapeDtypeStruct((NW, 2 * D), jnp.float32),
            jax.ShapeDtypeStruct((NW, 2 * L), jnp.int32),
        ),
        mesh=mesh,
        scratch_types=[
            pltpu.VMEM((CB * D,), jnp.float32),
            pltpu.VMEM((CB * D,), jnp.float32),
            pltpu.VMEM((CB,), jnp.int32),
            pltpu.VMEM((CB,), jnp.int32),
            pltpu.VMEM((D,), jnp.float32),
            pltpu.VMEM((D,), jnp.float32),
            pltpu.VMEM((2 * D,), jnp.float32),
            pltpu.VMEM((2 * L,), jnp.int32),
            pltpu.VMEM((32 * D,), jnp.float32),
            pltpu.SemaphoreType.DMA,
            pltpu.SemaphoreType.DMA,
        ],
    )
    def body(x_hbm, ids_hbm, dense_hbm, bsum_hbm, bid_hbm,
             xbuf0, xbuf1, idsbuf0, idsbuf1, stage, sumbuf, bbuf, bidbuf,
             zbuf, sem0, sem1):
        cid = lax.axis_index("c")
        sid = lax.axis_index("s")
        wid = sid * NC + cid
        zero = jnp.zeros((L,), jnp.float32)
        ones_i = jnp.full((L,), 1, jnp.int32)
        xbufs, idsbufs, sems = (xbuf0, xbuf1), (idsbuf0, idsbuf1), (sem0, sem1)

        for i in range(32 * NV):
            zbuf[pl.ds(i * L, L)] = zero
        pltpu.sync_copy(zbuf, dense_hbm.at[cid, pl.ds(sid * 32 * D, 32 * D)])

        for k in range(NV):
            bbuf[pl.ds(k * L, L)] = zero
        bidbuf[pl.ds(0, L)] = ones_i * -1

        plsc.subcore_barrier()

        base = wid * PER_W

        def win_of(s):
            return jnp.minimum(s, N_NODES - CB)

        def start_fetch(j, buf):
            win = win_of(base + j * CB)
            pltpu.async_copy(x_hbm.at[pl.ds(win * D, CB * D)],
                             xbufs[buf], sems[buf])
            pltpu.async_copy(ids_hbm.at[pl.ds(win, CB)],
                             idsbufs[buf], sems[buf])

        def wait_fetch(buf):
            pltpu.make_async_copy(x_hbm.at[pl.ds(0, CB * D)],
                                  xbufs[buf], sems[buf]).wait()
            pltpu.make_async_copy(ids_hbm.at[pl.ds(0, CB)],
                                  idsbufs[buf], sems[buf]).wait()

        def tree_add(vals):
            vals = list(vals)
            while len(vals) > 1:
                nxt = [vals[i] + vals[i + 1] for i in range(0, len(vals) - 1, 2)]
                if len(vals) % 2:
                    nxt.append(vals[-1])
                vals = nxt
            return vals[0]

        def row_step(rid, roff, xbuf, carry):
            cur, nf, sums = carry[0], carry[1], carry[2:]
            changed = rid != cur

            @pl.when(changed & (nf == 0))
            def _():
                for k in range(NV):
                    bbuf[pl.ds(k * L, L)] = sums[k]
                bidbuf[pl.ds(0, L)] = ones_i * cur

            @pl.when(changed & (nf >= 1))
            def _():
                for k in range(NV):
                    stage[pl.ds(k * L, L)] = sums[k]
                pltpu.sync_copy(stage, dense_hbm.at[cid, pl.ds(cur * D, D)])

            new_sums = tuple(
                jnp.where(changed, zero, sums[k])
                + xbuf[pl.ds(roff + k * L, L)]
                for k in range(NV)
            )
            nf = nf + changed.astype(jnp.int32)
            return (rid, nf) + new_sums

        def make_group_body(xbuf, idsbuf):
            def group_body(g, carry):
                idvec = idsbuf[pl.ds(g * L, L)]
                gbase = g * (L * D)
                same = (idvec[0] == idvec[L - 1]) & (idvec[0] == carry[0])

                def fast(c):
                    for r in range(L):
                        for k in range(NV):
                            plsc.addupdate(
                                sumbuf.at[pl.ds(k * L, L)],
                                xbuf[pl.ds(gbase + r * D + k * L, L)])
                    return c

                def slow(c):
                    full = c + tuple(sumbuf[pl.ds(k * L, L)]
                                     for k in range(NV))
                    for k in range(L):
                        full = row_step(idvec[k], gbase + k * D, xbuf, full)
                    for k in range(NV):
                        sumbuf[pl.ds(k * L, L)] = full[2 + k]
                    return full[:2]

                return lax.cond(same, fast, slow, carry)
            return group_body

        def process(j, buf, carry):
            s = base + j * CB
            win = win_of(s)
            g_lo = lax.div(s - win, L)
            g_hi = lax.div(jnp.minimum(s + CB, N_NODES) - win, L)
            return lax.fori_loop(g_lo, g_hi,
                                 make_group_body(xbufs[buf], idsbufs[buf]),
                                 carry)

        carry = (jnp.int32(-1), jnp.int32(-1))
        for k in range(NV):
            sumbuf[pl.ds(k * L, L)] = zero

        start_fetch(0, 0)

        def pair_body(p, carry):
            j = p * 2
            wait_fetch(0)
            start_fetch(j + 1, 1)
            carry = process(j, 0, carry)
            wait_fetch(1)
            start_fetch(j + 2, 0)
            carry = process(j + 1, 1, carry)
            return carry

        carry = lax.fori_loop(0, KMAX // 2, pair_body, carry)
        wait_fetch(0)

        cur = carry[0]
        for k in range(NV):
            bbuf[pl.ds(D + k * L, L)] = sumbuf[pl.ds(k * L, L)]
        bidbuf[pl.ds(L, L)] = ones_i * cur

        pltpu.sync_copy(bbuf, bsum_hbm.at[wid])
        pltpu.sync_copy(bidbuf, bid_hbm.at[wid])

    return body(x1d, seg)


def _head(pd, bsums, bids, W, b2):
    def body(pd_ref, bs_ref, bi_ref, w_ref, b_ref, o_ref):
        y = pd_ref[0] + pd_ref[1]
        bidv = bi_ref[...][:, 0]
        oh = (lax.broadcasted_iota(jnp.int32, (G, 2 * NW), 0)
              == bidv[None, :]).astype(jnp.float32)
        y = y + jnp.dot(oh, bs_ref[...], preferred_element_type=jnp.float32)
        y = jnp.maximum(y, 0.0)
        o_ref[...] = (
            jnp.dot(y, w_ref[...], preferred_element_type=jnp.float32)
            + b_ref[...]
        )

    return pl.pallas_call(
        body,
        out_shape=jax.ShapeDtypeStruct((G, NCLS), jnp.float32),
    )(pd, bsums, bids, W, b2)


def kernel(x, segment_ids, W, b):
    seg = segment_ids.astype(jnp.int32)
    pd, bsums, bids = _sc_segment_sum(x.reshape(-1), seg)
    return _head(pd.reshape(NC, G, D), bsums.reshape(2 * NW, D),
                 bids.reshape(2 * NW, L), W, b.reshape(1, NCLS))


# speedup vs baseline: 1.6612x; 1.6612x over previous
import functools

import jax
import jax.numpy as jnp
from jax import lax
from jax.experimental import pallas as pl
from jax.experimental.pallas import tpu as pltpu
from jax.experimental.pallas import tpu_sc as plsc

N_NODES = 100000
D = 128
G = 512
NCLS = 10
L = 16
NC = 2
NS = 16
NW = NC * NS
CB = 400
KMAX = 8
PER_W = CB * KMAX
NV = D // L


def _sc_segment_sum(x1d, seg):
    mesh = plsc.VectorSubcoreMesh(core_axis_name="c", subcore_axis_name="s")

    @functools.partial(
        pl.kernel,
        out_type=(
            jax.ShapeDtypeStruct((NC, G * D), jnp.float32),
            jax.ShapeDtypeStruct((NW, 2 * D), jnp.float32),
            jax.ShapeDtypeStruct((NW, 2 * L), jnp.int32),
        ),
        mesh=mesh,
        scratch_types=[
            pltpu.VMEM((CB * D,), jnp.float32),
            pltpu.VMEM((CB * D,), jnp.float32),
            pltpu.VMEM((CB,), jnp.int32),
            pltpu.VMEM((CB,), jnp.int32),
            pltpu.VMEM((D,), jnp.float32),
            pltpu.VMEM((D,), jnp.float32),
            pltpu.VMEM((2 * D,), jnp.float32),
            pltpu.VMEM((2 * L,), jnp.int32),
            pltpu.VMEM((32 * D,), jnp.float32),
            pltpu.SemaphoreType.DMA,
            pltpu.SemaphoreType.DMA,
        ],
    )
    def body(x_hbm, ids_hbm, dense_hbm, bsum_hbm, bid_hbm,
             xbuf0, xbuf1, idsbuf0, idsbuf1, stage, sumbuf, bbuf, bidbuf,
             zbuf, sem0, sem1):
        cid = lax.axis_index("c")
        sid = lax.axis_index("s")
        wid = sid * NC + cid
        zero = jnp.zeros((L,), jnp.float32)
        ones_i = jnp.full((L,), 1, jnp.int32)
        xbufs, idsbufs, sems = (xbuf0, xbuf1), (idsbuf0, idsbuf1), (sem0, sem1)

        for i in range(32 * NV):
            zbuf[pl.ds(i * L, L)] = zero
        pltpu.sync_copy(zbuf, dense_hbm.at[cid, pl.ds(sid * 32 * D, 32 * D)])

        for k in range(NV):
            bbuf[pl.ds(k * L, L)] = zero
        bidbuf[pl.ds(0, L)] = ones_i * -1

        plsc.subcore_barrier()

        base = wid * PER_W

        def win_of(s):
            return jnp.minimum(s, N_NODES - CB)

        def start_fetch(j, buf):
            win = win_of(base + j * CB)
            pltpu.async_copy(x_hbm.at[pl.ds(win * D, CB * D)],
                             xbufs[buf], sems[buf])
            pltpu.async_copy(ids_hbm.at[pl.ds(win, CB)],
                             idsbufs[buf], sems[buf])

        def wait_fetch(buf):
            pltpu.make_async_copy(x_hbm.at[pl.ds(0, CB * D)],
                                  xbufs[buf], sems[buf]).wait()
            pltpu.make_async_copy(ids_hbm.at[pl.ds(0, CB)],
                                  idsbufs[buf], sems[buf]).wait()

        def tree_add(vals):
            vals = list(vals)
            while len(vals) > 1:
                nxt = [vals[i] + vals[i + 1] for i in range(0, len(vals) - 1, 2)]
                if len(vals) % 2:
                    nxt.append(vals[-1])
                vals = nxt
            return vals[0]

        def row_step(rid, roff, xbuf, carry):
            cur, nf, sums = carry[0], carry[1], carry[2:]
            changed = rid != cur

            @pl.when(changed & (nf == 0))
            def _():
                for k in range(NV):
                    bbuf[pl.ds(k * L, L)] = sums[k]
                bidbuf[pl.ds(0, L)] = ones_i * cur

            @pl.when(changed & (nf >= 1))
            def _():
                for k in range(NV):
                    stage[pl.ds(k * L, L)] = sums[k]
                pltpu.sync_copy(stage, dense_hbm.at[cid, pl.ds(cur * D, D)])

            new_sums = tuple(
                jnp.where(changed, zero, sums[k])
                + xbuf[pl.ds(roff + k * L, L)]
                for k in range(NV)
            )
            nf = nf + changed.astype(jnp.int32)
            return (rid, nf) + new_sums

        def flush_to(nf, seg_id, vals):
            @pl.when(nf == 0)
            def _():
                for k in range(NV):
                    bbuf[pl.ds(k * L, L)] = vals[k]
                bidbuf[pl.ds(0, L)] = ones_i * seg_id

            @pl.when(nf >= 1)
            def _():
                for k in range(NV):
                    stage[pl.ds(k * L, L)] = vals[k]
                pltpu.sync_copy(stage, dense_hbm.at[cid, pl.ds(seg_id * D, D)])

        def make_group_body(xbuf, idsbuf):
            def group_body(g, carry):
                cur, nf = carry
                idvec = idsbuf[pl.ds(g * L, L)]
                gbase = g * (L * D)
                first = idvec[0]
                last = idvec[L - 1]
                same = (first == last) & (first == cur)

                nchg = (first != cur).astype(jnp.int32)
                for i in range(1, L):
                    nchg = nchg + (idvec[i] != idvec[i - 1]).astype(jnp.int32)

                @pl.when(same)
                def _():
                    for k in range(NV):
                        acc = tree_add(
                            [xbuf[pl.ds(gbase + r * D + k * L, L)]
                             for r in range(L)])
                        sumbuf[pl.ds(k * L, L)] = sumbuf[pl.ds(k * L, L)] + acc

                @pl.when(jnp.logical_not(same) & (nchg == 1))
                def _():
                    conts = [idvec[r] == cur for r in range(L)]
                    flushvals = []
                    for k in range(NV):
                        rows = [xbuf[pl.ds(gbase + r * D + k * L, L)]
                                for r in range(L)]
                        total = tree_add(rows)
                        part = tree_add([jnp.where(conts[r], rows[r], zero)
                                         for r in range(L)])
                        flushvals.append(sumbuf[pl.ds(k * L, L)] + part)
                        sumbuf[pl.ds(k * L, L)] = total - part
                    flush_to(nf, cur, flushvals)

                @pl.when(nchg >= 2)
                def _():
                    full = (cur, nf) + tuple(sumbuf[pl.ds(k * L, L)]
                                             for k in range(NV))
                    for k in range(L):
                        full = row_step(idvec[k], gbase + k * D, xbuf, full)
                    for k in range(NV):
                        sumbuf[pl.ds(k * L, L)] = full[2 + k]

                return (last, nf + nchg)
            return group_body

        def process(j, buf, carry):
            s = base + j * CB
            win = win_of(s)
            g_lo = lax.div(s - win, L)
            g_hi = lax.div(jnp.minimum(s + CB, N_NODES) - win, L)
            return lax.fori_loop(g_lo, g_hi,
                                 make_group_body(xbufs[buf], idsbufs[buf]),
                                 carry)

        carry = (jnp.int32(-1), jnp.int32(-1))
        for k in range(NV):
            sumbuf[pl.ds(k * L, L)] = zero

        start_fetch(0, 0)

        def pair_body(p, carry):
            j = p * 2
            wait_fetch(0)
            start_fetch(j + 1, 1)
            carry = process(j, 0, carry)
            wait_fetch(1)
            start_fetch(j + 2, 0)
            carry = process(j + 1, 1, carry)
            return carry

        carry = lax.fori_loop(0, KMAX // 2, pair_body, carry)
        wait_fetch(0)

        cur = carry[0]
        for k in range(NV):
            bbuf[pl.ds(D + k * L, L)] = sumbuf[pl.ds(k * L, L)]
        bidbuf[pl.ds(L, L)] = ones_i * cur

        pltpu.sync_copy(bbuf, bsum_hbm.at[wid])
        pltpu.sync_copy(bidbuf, bid_hbm.at[wid])

    return body(x1d, seg)


def _head(pd, bsums, bids, W, b2):
    def body(pd_ref, bs_ref, bi_ref, w_ref, b_ref, o_ref):
        y = pd_ref[0] + pd_ref[1]
        bidv = bi_ref[...][:, 0]
        oh = (lax.broadcasted_iota(jnp.int32, (G, 2 * NW), 0)
              == bidv[None, :]).astype(jnp.float32)
        y = y + jnp.dot(oh, bs_ref[...], preferred_element_type=jnp.float32)
        y = jnp.maximum(y, 0.0)
        o_ref[...] = (
            jnp.dot(y, w_ref[...], preferred_element_type=jnp.float32)
            + b_ref[...]
        )

    return pl.pallas_call(
        body,
        out_shape=jax.ShapeDtypeStruct((G, NCLS), jnp.float32),
    )(pd, bsums, bids, W, b2)


def kernel(x, segment_ids, W, b):
    seg = segment_ids.astype(jnp.int32)
    pd, bsums, bids = _sc_segment_sum(x.reshape(-1), seg)
    return _head(pd.reshape(NC, G, D), bsums.reshape(2 * NW, D),
                 bids.reshape(2 * NW, L), W, b.reshape(1, NCLS))


# speedup vs baseline: 1.7501x; 1.0536x over previous
import functools

import jax
import jax.numpy as jnp
from jax import lax
from jax.experimental import pallas as pl
from jax.experimental.pallas import tpu as pltpu
from jax.experimental.pallas import tpu_sc as plsc

N_NODES = 100000
D = 128
G = 512
NCLS = 10
L = 16
NC = 2
NS = 16
NW = NC * NS
CB = 400
KMAX = 8
PER_W = CB * KMAX
NV = D // L


def _sc_segment_sum(x1d, seg):
    mesh = plsc.VectorSubcoreMesh(core_axis_name="c", subcore_axis_name="s")

    @functools.partial(
        pl.kernel,
        out_type=(
            jax.ShapeDtypeStruct((NC, G * D), jnp.float32),
            jax.ShapeDtypeStruct((NW, 2 * D), jnp.float32),
            jax.ShapeDtypeStruct((NW, 2 * L), jnp.int32),
        ),
        mesh=mesh,
        scratch_types=[
            pltpu.VMEM((CB * D,), jnp.float32),
            pltpu.VMEM((CB * D,), jnp.float32),
            pltpu.VMEM((CB,), jnp.int32),
            pltpu.VMEM((CB,), jnp.int32),
            pltpu.VMEM((D,), jnp.float32),
            pltpu.VMEM((D,), jnp.float32),
            pltpu.VMEM((2 * D,), jnp.float32),
            pltpu.VMEM((2 * L,), jnp.int32),
            pltpu.VMEM((32 * D,), jnp.float32),
            pltpu.SMEM((1,), jnp.int32),
            pltpu.SemaphoreType.DMA,
            pltpu.SemaphoreType.DMA,
        ],
    )
    def body(x_hbm, ids_hbm, dense_hbm, bsum_hbm, bid_hbm,
             xbuf0, xbuf1, idsbuf0, idsbuf1, stage, sumbuf, bbuf, bidbuf,
             zbuf, flagbuf, sem0, sem1):
        cid = lax.axis_index("c")
        sid = lax.axis_index("s")
        wid = sid * NC + cid
        zero = jnp.zeros((L,), jnp.float32)
        ones_i = jnp.full((L,), 1, jnp.int32)
        xbufs, idsbufs, sems = (xbuf0, xbuf1), (idsbuf0, idsbuf1), (sem0, sem1)

        for i in range(32 * NV):
            zbuf[pl.ds(i * L, L)] = zero
        pltpu.sync_copy(zbuf, dense_hbm.at[cid, pl.ds(sid * 32 * D, 32 * D)])

        for k in range(NV):
            bbuf[pl.ds(k * L, L)] = zero
        bidbuf[pl.ds(0, L)] = ones_i * -1
        flagbuf[0] = 0

        plsc.subcore_barrier()

        base = wid * PER_W

        def win_of(s):
            return jnp.minimum(s, N_NODES - CB)

        def start_fetch(j, buf):
            win = win_of(base + j * CB)
            pltpu.async_copy(x_hbm.at[pl.ds(win * D, CB * D)],
                             xbufs[buf], sems[buf])
            pltpu.async_copy(ids_hbm.at[pl.ds(win, CB)],
                             idsbufs[buf], sems[buf])

        def wait_fetch(buf):
            pltpu.make_async_copy(x_hbm.at[pl.ds(0, CB * D)],
                                  xbufs[buf], sems[buf]).wait()
            pltpu.make_async_copy(ids_hbm.at[pl.ds(0, CB)],
                                  idsbufs[buf], sems[buf]).wait()

        def tree_add(vals):
            vals = list(vals)
            while len(vals) > 1:
                nxt = [vals[i] + vals[i + 1] for i in range(0, len(vals) - 1, 2)]
                if len(vals) % 2:
                    nxt.append(vals[-1])
                vals = nxt
            return vals[0]

        def flush_to(seg_id, vals):
            f = flagbuf[0]

            @pl.when(f == 0)
            def _():
                flagbuf[0] = 1

            @pl.when(f == 1)
            def _():
                for k in range(NV):
                    bbuf[pl.ds(k * L, L)] = vals[k]
                bidbuf[pl.ds(0, L)] = ones_i * seg_id
                flagbuf[0] = 2

            @pl.when(f == 2)
            def _():
                for k in range(NV):
                    stage[pl.ds(k * L, L)] = vals[k]
                pltpu.sync_copy(stage, dense_hbm.at[cid, pl.ds(seg_id * D, D)])

        def row_step(rid, roff, xbuf, carry):
            cur, sums = carry[0], carry[1:]
            changed = rid != cur

            @pl.when(changed)
            def _():
                flush_to(cur, sums)

            new_sums = tuple(
                jnp.where(changed, zero, sums[k])
                + xbuf[pl.ds(roff + k * L, L)]
                for k in range(NV)
            )
            return (rid,) + new_sums

        def make_group_body(xbuf, idsbuf):
            def group_body(g, cur):
                idvec = idsbuf[pl.ds(g * L, L)]
                gbase = g * (L * D)
                first = idvec[0]
                last = idvec[L - 1]
                same = (first == last) & (first == cur)
                nchg = jnp.int32(0)
                for i in range(1, L):
                    nchg = nchg + (idvec[i] != idvec[i - 1]).astype(jnp.int32)
                two = nchg == 1
                mask_ok = jnp.where(first == cur, two, first == last)

                @pl.when(same)
                def _():
                    for k in range(NV):
                        acc = tree_add(
                            [xbuf[pl.ds(gbase + r * D + k * L, L)]
                             for r in range(L)])
                        sumbuf[pl.ds(k * L, L)] = sumbuf[pl.ds(k * L, L)] + acc

                @pl.when(jnp.logical_not(same) & mask_ok)
                def _():
                    conts = [idvec[r] == cur for r in range(L)]
                    flushvals = []
                    for k in range(NV):
                        rows = [xbuf[pl.ds(gbase + r * D + k * L, L)]
                                for r in range(L)]
                        total = tree_add(rows)
                        part = tree_add([jnp.where(conts[r], rows[r], zero)
                                         for r in range(L)])
                        flushvals.append(sumbuf[pl.ds(k * L, L)] + part)
                        sumbuf[pl.ds(k * L, L)] = total - part
                    flush_to(cur, flushvals)

                @pl.when(jnp.logical_not(same) & jnp.logical_not(mask_ok))
                def _():
                    full = (cur,) + tuple(sumbuf[pl.ds(k * L, L)]
                                          for k in range(NV))
                    for k in range(L):
                        full = row_step(idvec[k], gbase + k * D, xbuf, full)
                    for k in range(NV):
                        sumbuf[pl.ds(k * L, L)] = full[1 + k]

                return last
            return group_body

        def process(j, buf, carry):
            s = base + j * CB
            win = win_of(s)
            g_lo = lax.div(s - win, L)
            g_hi = lax.div(jnp.minimum(s + CB, N_NODES) - win, L)
            return lax.fori_loop(g_lo, g_hi,
                                 make_group_body(xbufs[buf], idsbufs[buf]),
                                 carry)

        carry = jnp.int32(-1)
        for k in range(NV):
            sumbuf[pl.ds(k * L, L)] = zero

        start_fetch(0, 0)

        def pair_body(p, carry):
            j = p * 2
            wait_fetch(0)
            start_fetch(j + 1, 1)
            carry = process(j, 0, carry)
            wait_fetch(1)
            start_fetch(j + 2, 0)
            carry = process(j + 1, 1, carry)
            return carry

        carry = lax.fori_loop(0, KMAX // 2, pair_body, carry)
        wait_fetch(0)

        cur = carry
        for k in range(NV):
            bbuf[pl.ds(D + k * L, L)] = sumbuf[pl.ds(k * L, L)]
        bidbuf[pl.ds(L, L)] = ones_i * cur

        pltpu.sync_copy(bbuf, bsum_hbm.at[wid])
        pltpu.sync_copy(bidbuf, bid_hbm.at[wid])

    return body(x1d, seg)


def _head(pd, bsums, bids, W, b2):
    def body(pd_ref, bs_ref, bi_ref, w_ref, b_ref, o_ref):
        y = pd_ref[0] + pd_ref[1]
        bidv = bi_ref[...][:, 0]
        oh = (lax.broadcasted_iota(jnp.int32, (G, 2 * NW), 0)
              == bidv[None, :]).astype(jnp.float32)
        y = y + jnp.dot(oh, bs_ref[...], preferred_element_type=jnp.float32)
        y = jnp.maximum(y, 0.0)
        o_ref[...] = (
            jnp.dot(y, w_ref[...], preferred_element_type=jnp.float32)
            + b_ref[...]
        )

    return pl.pallas_call(
        body,
        out_shape=jax.ShapeDtypeStruct((G, NCLS), jnp.float32),
    )(pd, bsums, bids, W, b2)


def kernel(x, segment_ids, W, b):
    seg = segment_ids.astype(jnp.int32)
    pd, bsums, bids = _sc_segment_sum(x.reshape(-1), seg)
    return _head(pd.reshape(NC, G, D), bsums.reshape(2 * NW, D),
                 bids.reshape(2 * NW, L), W, b.reshape(1, NCLS))


# speedup vs baseline: 1.7576x; 1.0042x over previous
import functools

import jax
import jax.numpy as jnp
from jax import lax
from jax.experimental import pallas as pl
from jax.experimental.pallas import tpu as pltpu
from jax.experimental.pallas import tpu_sc as plsc

N_NODES = 100000
D = 128
G = 512
NCLS = 10
L = 16
NC = 2
NS = 16
NW = NC * NS
CB = 400
KMAX = 8
PER_W = CB * KMAX
NV = D // L


def _sc_segment_sum(x1d, seg):
    mesh = plsc.VectorSubcoreMesh(core_axis_name="c", subcore_axis_name="s")

    @functools.partial(
        pl.kernel,
        out_type=(
            jax.ShapeDtypeStruct((NC, G * D), jnp.float32),
            jax.ShapeDtypeStruct((NW, 2 * D), jnp.float32),
            jax.ShapeDtypeStruct((NW, 2 * L), jnp.int32),
        ),
        mesh=mesh,
        scratch_types=[
            pltpu.VMEM((CB * D,), jnp.float32),
            pltpu.VMEM((CB * D,), jnp.float32),
            pltpu.VMEM((CB,), jnp.int32),
            pltpu.VMEM((CB,), jnp.int32),
            pltpu.VMEM((D,), jnp.float32),
            pltpu.VMEM((D,), jnp.float32),
            pltpu.VMEM((2 * D,), jnp.float32),
            pltpu.VMEM((2 * L,), jnp.int32),
            pltpu.VMEM((32 * D,), jnp.float32),
            pltpu.SMEM((1,), jnp.int32),
            pltpu.SemaphoreType.DMA,
            pltpu.SemaphoreType.DMA,
        ],
    )
    def body(x_hbm, ids_hbm, dense_hbm, bsum_hbm, bid_hbm,
             xbuf0, xbuf1, idsbuf0, idsbuf1, stage, sumbuf, bbuf, bidbuf,
             zbuf, flagbuf, sem0, sem1):
        cid = lax.axis_index("c")
        sid = lax.axis_index("s")
        wid = sid * NC + cid
        zero = jnp.zeros((L,), jnp.float32)
        ones_i = jnp.full((L,), 1, jnp.int32)
        xbufs, idsbufs, sems = (xbuf0, xbuf1), (idsbuf0, idsbuf1), (sem0, sem1)

        for i in range(32 * NV):
            zbuf[pl.ds(i * L, L)] = zero
        pltpu.sync_copy(zbuf, dense_hbm.at[cid, pl.ds(sid * 32 * D, 32 * D)])

        for k in range(NV):
            bbuf[pl.ds(k * L, L)] = zero
        bidbuf[pl.ds(0, L)] = ones_i * -1
        flagbuf[0] = 0

        plsc.subcore_barrier()

        base = wid * PER_W

        def win_of(s):
            return jnp.minimum(s, N_NODES - CB)

        def start_fetch(j, buf):
            win = win_of(base + j * CB)
            pltpu.async_copy(x_hbm.at[pl.ds(win * D, CB * D)],
                             xbufs[buf], sems[buf])
            pltpu.async_copy(ids_hbm.at[pl.ds(win, CB)],
                             idsbufs[buf], sems[buf])

        def wait_fetch(buf):
            pltpu.make_async_copy(x_hbm.at[pl.ds(0, CB * D)],
                                  xbufs[buf], sems[buf]).wait()
            pltpu.make_async_copy(ids_hbm.at[pl.ds(0, CB)],
                                  idsbufs[buf], sems[buf]).wait()

        def tree_add(vals):
            vals = list(vals)
            while len(vals) > 1:
                nxt = [vals[i] + vals[i + 1] for i in range(0, len(vals) - 1, 2)]
                if len(vals) % 2:
                    nxt.append(vals[-1])
                vals = nxt
            return vals[0]

        def flush_to(seg_id, vals):
            f = flagbuf[0]

            @pl.when(f == 0)
            def _():
                flagbuf[0] = 1

            @pl.when(f == 1)
            def _():
                for k in range(NV):
                    bbuf[pl.ds(k * L, L)] = vals[k]
                bidbuf[pl.ds(0, L)] = ones_i * seg_id
                flagbuf[0] = 2

            @pl.when(f == 2)
            def _():
                for k in range(NV):
                    stage[pl.ds(k * L, L)] = vals[k]
                pltpu.sync_copy(stage, dense_hbm.at[cid, pl.ds(seg_id * D, D)])

        def row_step(rid, roff, xbuf, carry):
            cur, sums = carry[0], carry[1:]
            changed = rid != cur

            @pl.when(changed)
            def _():
                flush_to(cur, sums)

            new_sums = tuple(
                jnp.where(changed, zero, sums[k])
                + xbuf[pl.ds(roff + k * L, L)]
                for k in range(NV)
            )
            return (rid,) + new_sums

        def make_group_body(xbuf, idsbuf):
            def group_body(g, cur):
                idvec = idsbuf[pl.ds(g * L, L)]
                gbase = g * (L * D)
                first = idvec[0]
                last = idvec[L - 1]
                same = (first == last) & (first == cur)

                @pl.when(same)
                def _():
                    for k in range(NV):
                        acc = tree_add(
                            [xbuf[pl.ds(gbase + r * D + k * L, L)]
                             for r in range(L)])
                        sumbuf[pl.ds(k * L, L)] = sumbuf[pl.ds(k * L, L)] + acc

                @pl.when(jnp.logical_not(same))
                def _():
                    nchg = jnp.int32(0)
                    for i in range(1, L):
                        nchg = nchg + (idvec[i] != idvec[i - 1]).astype(
                            jnp.int32)
                    mask_ok = jnp.where(first == cur, nchg == 1,
                                        first == last)

                    @pl.when(mask_ok)
                    def _():
                        conts = [idvec[r] == cur for r in range(L)]
                        flushvals = []
                        for k in range(NV):
                            rows = [xbuf[pl.ds(gbase + r * D + k * L, L)]
                                    for r in range(L)]
                            total = tree_add(rows)
                            part = tree_add(
                                [jnp.where(conts[r], rows[r], zero)
                                 for r in range(L)])
                            flushvals.append(sumbuf[pl.ds(k * L, L)] + part)
                            sumbuf[pl.ds(k * L, L)] = total - part
                        flush_to(cur, flushvals)

                    @pl.when(jnp.logical_not(mask_ok))
                    def _():
                        full = (cur,) + tuple(sumbuf[pl.ds(k * L, L)]
                                              for k in range(NV))
                        for k in range(L):
                            full = row_step(idvec[k], gbase + k * D,
                                            xbuf, full)
                        for k in range(NV):
                            sumbuf[pl.ds(k * L, L)] = full[1 + k]

                return last
            return group_body

        def process(j, buf, carry):
            s = base + j * CB
            win = win_of(s)
            g_lo = lax.div(s - win, L)
            g_hi = lax.div(jnp.minimum(s + CB, N_NODES) - win, L)
            return lax.fori_loop(g_lo, g_hi,
                                 make_group_body(xbufs[buf], idsbufs[buf]),
                                 carry)

        carry = jnp.int32(-1)
        for k in range(NV):
            sumbuf[pl.ds(k * L, L)] = zero

        start_fetch(0, 0)

        def pair_body(p, carry):
            j = p * 2
            wait_fetch(0)
            start_fetch(j + 1, 1)
            carry = process(j, 0, carry)
            wait_fetch(1)
            start_fetch(j + 2, 0)
            carry = process(j + 1, 1, carry)
            return carry

        carry = lax.fori_loop(0, KMAX // 2, pair_body, carry)
        wait_fetch(0)

        cur = carry
        for k in range(NV):
            bbuf[pl.ds(D + k * L, L)] = sumbuf[pl.ds(k * L, L)]
        bidbuf[pl.ds(L, L)] = ones_i * cur

        pltpu.sync_copy(bbuf, bsum_hbm.at[wid])
        pltpu.sync_copy(bidbuf, bid_hbm.at[wid])

    return body(x1d, seg)


def _head(pd, bsums, bids, W, b2):
    def body(pd_ref, bs_ref, bi_ref, w_ref, b_ref, o_ref):
        y = pd_ref[0] + pd_ref[1]
        bidv = bi_ref[...][:, 0]
        oh = (lax.broadcasted_iota(jnp.int32, (G, 2 * NW), 0)
              == bidv[None, :]).astype(jnp.float32)
        y = y + jnp.dot(oh, bs_ref[...], preferred_element_type=jnp.float32)
        y = jnp.maximum(y, 0.0)
        o_ref[...] = (
            jnp.dot(y, w_ref[...], preferred_element_type=jnp.float32)
            + b_ref[...]
        )

    return pl.pallas_call(
        body,
        out_shape=jax.ShapeDtypeStruct((G, NCLS), jnp.float32),
    )(pd, bsums, bids, W, b2)


def kernel(x, segment_ids, W, b):
    seg = segment_ids.astype(jnp.int32)
    pd, bsums, bids = _sc_segment_sum(x.reshape(-1), seg)
    return _head(pd.reshape(NC, G, D), bsums.reshape(2 * NW, D),
                 bids.reshape(2 * NW, L), W, b.reshape(1, NCLS))


# speedup vs baseline: 1.8675x; 1.0625x over previous
import functools

import jax
import jax.numpy as jnp
from jax import lax
from jax.experimental import pallas as pl
from jax.experimental.pallas import tpu as pltpu
from jax.experimental.pallas import tpu_sc as plsc

N_NODES = 100000
D = 128
G = 512
NCLS = 10
L = 16
NC = 2
NS = 16
NW = NC * NS
CB = 400
KMAX = 8
PER_W = CB * KMAX
NV = D // L


def _sc_segment_sum(x1d, seg):
    mesh = plsc.VectorSubcoreMesh(core_axis_name="c", subcore_axis_name="s")

    @functools.partial(
        pl.kernel,
        out_type=(
            jax.ShapeDtypeStruct((NC, G * D), jnp.float32),
            jax.ShapeDtypeStruct((NW, 2 * D), jnp.float32),
            jax.ShapeDtypeStruct((NW, 2 * L), jnp.int32),
        ),
        mesh=mesh,
        scratch_types=[
            pltpu.VMEM((2 * CB * D,), jnp.float32),
            pltpu.VMEM((2 * CB + L,), jnp.int32),
            pltpu.VMEM((D,), jnp.float32),
            pltpu.VMEM((D,), jnp.float32),
            pltpu.VMEM((2 * D,), jnp.float32),
            pltpu.VMEM((2 * L,), jnp.int32),
            pltpu.VMEM((32 * D,), jnp.float32),
            pltpu.SMEM((1,), jnp.int32),
            pltpu.SemaphoreType.DMA,
            pltpu.SemaphoreType.DMA,
        ],
    )
    def body(x_hbm, ids_hbm, dense_hbm, bsum_hbm, bid_hbm,
             xbuf, idsbuf, stage, sumbuf, bbuf, bidbuf,
             zbuf, flagbuf, sem0, sem1):
        cid = lax.axis_index("c")
        sid = lax.axis_index("s")
        wid = sid * NC + cid
        zero = jnp.zeros((L,), jnp.float32)
        ones_i = jnp.full((L,), 1, jnp.int32)
        sems = (sem0, sem1)

        for i in range(32 * NV):
            zbuf[pl.ds(i * L, L)] = zero
        pltpu.sync_copy(zbuf, dense_hbm.at[cid, pl.ds(sid * 32 * D, 32 * D)])

        for k in range(NV):
            bbuf[pl.ds(k * L, L)] = zero
        bidbuf[pl.ds(0, L)] = ones_i * -1
        flagbuf[0] = 0

        plsc.subcore_barrier()

        base = wid * PER_W

        def win_of(s):
            return jnp.minimum(s, N_NODES - CB)

        def start_fetch(j, buf):
            win = win_of(base + j * CB)
            pltpu.async_copy(x_hbm.at[pl.ds(win * D, CB * D)],
                             xbuf.at[pl.ds(buf * (CB * D), CB * D)],
                             sems[buf])
            pltpu.async_copy(ids_hbm.at[pl.ds(win, CB)],
                             idsbuf.at[pl.ds(buf * CB, CB)], sems[buf])

        def wait_fetch(buf):
            pltpu.make_async_copy(x_hbm.at[pl.ds(0, CB * D)],
                                  xbuf.at[pl.ds(buf * (CB * D), CB * D)],
                                  sems[buf]).wait()
            pltpu.make_async_copy(ids_hbm.at[pl.ds(0, CB)],
                                  idsbuf.at[pl.ds(buf * CB, CB)],
                                  sems[buf]).wait()

        def tree_add(vals):
            vals = list(vals)
            while len(vals) > 1:
                nxt = [vals[i] + vals[i + 1] for i in range(0, len(vals) - 1, 2)]
                if len(vals) % 2:
                    nxt.append(vals[-1])
                vals = nxt
            return vals[0]

        def flush_to(seg_id, vals):
            f = flagbuf[0]

            @pl.when(f == 0)
            def _():
                flagbuf[0] = 1

            @pl.when(f == 1)
            def _():
                for k in range(NV):
                    bbuf[pl.ds(k * L, L)] = vals[k]
                bidbuf[pl.ds(0, L)] = ones_i * seg_id
                flagbuf[0] = 2

            @pl.when(f == 2)
            def _():
                for k in range(NV):
                    stage[pl.ds(k * L, L)] = vals[k]
                pltpu.sync_copy(stage, dense_hbm.at[cid, pl.ds(seg_id * D, D)])

        def group_body(roff0, g, cur):
            gro = roff0 + g * L
            idvec = idsbuf[pl.ds(gro, L)]
            gbase = gro * D
            first = idvec[0]
            last = idvec[L - 1]
            same = (first == last) & (first == cur)

            @pl.when(same)
            def _():
                for k in range(NV):
                    acc = tree_add(
                        [xbuf[pl.ds(gbase + r * D + k * L, L)]
                         for r in range(L)])
                    sumbuf[pl.ds(k * L, L)] = sumbuf[pl.ds(k * L, L)] + acc

            @pl.when(jnp.logical_not(same))
            def _():
                nchg = jnp.int32(0)
                for i in range(1, L):
                    nchg = nchg + (idvec[i] != idvec[i - 1]).astype(jnp.int32)
                mask_ok = jnp.where(first == cur, nchg == 1, first == last)

                @pl.when(mask_ok)
                def _():
                    conts = [idvec[r] == cur for r in range(L)]
                    flushvals = []
                    for k in range(NV):
                        rows = [xbuf[pl.ds(gbase + r * D + k * L, L)]
                                for r in range(L)]
                        total = tree_add(rows)
                        part = tree_add([jnp.where(conts[r], rows[r], zero)
                                         for r in range(L)])
                        flushvals.append(sumbuf[pl.ds(k * L, L)] + part)
                        sumbuf[pl.ds(k * L, L)] = total - part
                    flush_to(cur, flushvals)

                @pl.when(jnp.logical_not(mask_ok))
                def _():
                    def row_step(r, full):
                        cur_r, sums = full[0], full[1:]
                        rid = idsbuf[pl.ds(gro + r, L)][0]
                        changed = rid != cur_r

                        @pl.when(changed)
                        def _():
                            flush_to(cur_r, sums)

                        new_sums = tuple(
                            jnp.where(changed, zero, sums[k])
                            + xbuf[pl.ds(gbase + r * D + k * L, L)]
                            for k in range(NV)
                        )
                        return (rid,) + new_sums

                    full = (cur,) + tuple(sumbuf[pl.ds(k * L, L)]
                                          for k in range(NV))
                    full = lax.fori_loop(0, L, row_step, full)
                    for k in range(NV):
                        sumbuf[pl.ds(k * L, L)] = full[1 + k]

            return last

        carry = jnp.int32(-1)
        for k in range(NV):
            sumbuf[pl.ds(k * L, L)] = zero

        start_fetch(0, 0)

        def chunk_body(j, carry):
            par = lax.rem(j, 2)

            @pl.when(par == 0)
            def _():
                wait_fetch(0)
                start_fetch(j + 1, 1)

            @pl.when(par == 1)
            def _():
                wait_fetch(1)
                start_fetch(j + 1, 0)

            s = base + j * CB
            win = win_of(s)
            g_lo = lax.div(s - win, L)
            g_hi = lax.div(jnp.minimum(s + CB, N_NODES) - win, L)
            roff0 = par * CB
            return lax.fori_loop(
                g_lo, g_hi,
                lambda g, cur: group_body(roff0, g, cur), carry)

        carry = lax.fori_loop(0, KMAX, chunk_body, carry)
        wait_fetch(0)

        cur = carry
        for k in range(NV):
            bbuf[pl.ds(D + k * L, L)] = sumbuf[pl.ds(k * L, L)]
        bidbuf[pl.ds(L, L)] = ones_i * cur

        pltpu.sync_copy(bbuf, bsum_hbm.at[wid])
        pltpu.sync_copy(bidbuf, bid_hbm.at[wid])

    return body(x1d, seg)


def _head(pd, bsums, bids, W, b2):
    def body(pd_ref, bs_ref, bi_ref, w_ref, b_ref, o_ref):
        y = pd_ref[0] + pd_ref[1]
        bidv = bi_ref[...][:, 0]
        oh = (lax.broadcasted_iota(jnp.int32, (G, 2 * NW), 0)
              == bidv[None, :]).astype(jnp.float32)
        y = y + jnp.dot(oh, bs_ref[...], preferred_element_type=jnp.float32)
        y = jnp.maximum(y, 0.0)
        o_ref[...] = (
            jnp.dot(y, w_ref[...], preferred_element_type=jnp.float32)
            + b_ref[...]
        )

    return pl.pallas_call(
        body,
        out_shape=jax.ShapeDtypeStruct((G, NCLS), jnp.float32),
    )(pd, bsums, bids, W, b2)


def kernel(x, segment_ids, W, b):
    seg = segment_ids.astype(jnp.int32)
    pd, bsums, bids = _sc_segment_sum(x.reshape(-1), seg)
    return _head(pd.reshape(NC, G, D), bsums.reshape(2 * NW, D),
                 bids.reshape(2 * NW, L), W, b.reshape(1, NCLS))


# speedup vs baseline: 1.9407x; 1.0392x over previous
import functools

import jax
import jax.numpy as jnp
from jax import lax
from jax.experimental import pallas as pl
from jax.experimental.pallas import tpu as pltpu
from jax.experimental.pallas import tpu_sc as plsc

N_NODES = 100000
D = 128
G = 512
NCLS = 10
L = 16
NC = 2
NS = 16
NW = NC * NS
CB = 400
KMAX = 8
PER_W = CB * KMAX
NV = D // L


def _sc_segment_sum(x1d, seg):
    mesh = plsc.VectorSubcoreMesh(core_axis_name="c", subcore_axis_name="s")

    @functools.partial(
        pl.kernel,
        out_type=(
            jax.ShapeDtypeStruct((NC, G * D), jnp.float32),
            jax.ShapeDtypeStruct((NW, 2 * D), jnp.float32),
            jax.ShapeDtypeStruct((NW, 2 * L), jnp.int32),
        ),
        mesh=mesh,
        scratch_types=[
            pltpu.VMEM((2 * CB * D,), jnp.float32),
            pltpu.VMEM((2 * CB + L,), jnp.int32),
            pltpu.VMEM((D,), jnp.float32),
            pltpu.VMEM((D,), jnp.float32),
            pltpu.VMEM((2 * D,), jnp.float32),
            pltpu.VMEM((2 * L,), jnp.int32),
            pltpu.VMEM((32 * D,), jnp.float32),
            pltpu.SMEM((1,), jnp.int32),
            pltpu.SemaphoreType.DMA,
            pltpu.SemaphoreType.DMA,
        ],
    )
    def body(x_hbm, ids_hbm, dense_hbm, bsum_hbm, bid_hbm,
             xbuf, idsbuf, stage, sumbuf, bbuf, bidbuf,
             zbuf, flagbuf, sem0, sem1):
        cid = lax.axis_index("c")
        sid = lax.axis_index("s")
        wid = sid * NC + cid
        zero = jnp.zeros((L,), jnp.float32)
        ones_i = jnp.full((L,), 1, jnp.int32)
        sems = (sem0, sem1)

        for i in range(32 * NV):
            zbuf[pl.ds(i * L, L)] = zero
        pltpu.sync_copy(zbuf, dense_hbm.at[cid, pl.ds(sid * 32 * D, 32 * D)])

        for k in range(NV):
            bbuf[pl.ds(k * L, L)] = zero
        bidbuf[pl.ds(0, L)] = ones_i * -1
        flagbuf[0] = 0

        plsc.subcore_barrier()

        base = wid * PER_W

        def win_of(s):
            return jnp.minimum(s, N_NODES - CB)

        def start_fetch(j, buf):
            win = win_of(base + j * CB)
            pltpu.async_copy(x_hbm.at[pl.ds(win * D, CB * D)],
                             xbuf.at[pl.ds(buf * (CB * D), CB * D)],
                             sems[buf])
            pltpu.async_copy(ids_hbm.at[pl.ds(win, CB)],
                             idsbuf.at[pl.ds(buf * CB, CB)], sems[buf])

        def wait_fetch(buf):
            pltpu.make_async_copy(x_hbm.at[pl.ds(0, CB * D)],
                                  xbuf.at[pl.ds(buf * (CB * D), CB * D)],
                                  sems[buf]).wait()
            pltpu.make_async_copy(ids_hbm.at[pl.ds(0, CB)],
                                  idsbuf.at[pl.ds(buf * CB, CB)],
                                  sems[buf]).wait()

        def tree_add(vals):
            vals = list(vals)
            while len(vals) > 1:
                nxt = [vals[i] + vals[i + 1] for i in range(0, len(vals) - 1, 2)]
                if len(vals) % 2:
                    nxt.append(vals[-1])
                vals = nxt
            return vals[0]

        def flush_to(seg_id, vals):
            f = flagbuf[0]

            @pl.when(f == 0)
            def _():
                flagbuf[0] = 1

            @pl.when(f == 1)
            def _():
                for k in range(NV):
                    bbuf[pl.ds(k * L, L)] = vals[k]
                bidbuf[pl.ds(0, L)] = ones_i * seg_id
                flagbuf[0] = 2

            @pl.when(f == 2)
            def _():
                for k in range(NV):
                    stage[pl.ds(k * L, L)] = vals[k]
                pltpu.sync_copy(stage, dense_hbm.at[cid, pl.ds(seg_id * D, D)])

        def group_body(roff0, g, cur):
            gro = roff0 + g * L
            idvec = idsbuf[pl.ds(gro, L)]
            gbase = gro * D
            first = idvec[0]
            last = idvec[L - 1]
            same = (first == last) & (first == cur)

            @pl.when(same)
            def _():
                for k in range(NV):
                    acc = tree_add(
                        [xbuf[pl.ds(gbase + r * D + k * L, L)]
                         for r in range(L)])
                    sumbuf[pl.ds(k * L, L)] = sumbuf[pl.ds(k * L, L)] + acc

            @pl.when(jnp.logical_not(same))
            def _():
                nchg = jnp.int32(0)
                for i in range(1, L):
                    nchg = nchg + (idvec[i] != idvec[i - 1]).astype(jnp.int32)
                mask_ok = jnp.where(first == cur, nchg == 1, first == last)

                @pl.when(mask_ok)
                def _():
                    conts = [idvec[r] == cur for r in range(L)]
                    flushvals = []
                    for k in range(NV):
                        rows = [xbuf[pl.ds(gbase + r * D + k * L, L)]
                                for r in range(L)]
                        total = tree_add(rows)
                        part = tree_add([jnp.where(conts[r], rows[r], zero)
                                         for r in range(L)])
                        flushvals.append(sumbuf[pl.ds(k * L, L)] + part)
                        sumbuf[pl.ds(k * L, L)] = total - part
                    flush_to(cur, flushvals)

                @pl.when(jnp.logical_not(mask_ok))
                def _():
                    def row_step(r, full):
                        cur_r, sums = full[0], full[1:]
                        rid = idsbuf[pl.ds(gro + r, L)][0]
                        changed = rid != cur_r

                        @pl.when(changed)
                        def _():
                            flush_to(cur_r, sums)

                        new_sums = tuple(
                            jnp.where(changed, zero, sums[k])
                            + xbuf[pl.ds(gbase + r * D + k * L, L)]
                            for k in range(NV)
                        )
                        return (rid,) + new_sums

                    full = (cur,) + tuple(sumbuf[pl.ds(k * L, L)]
                                          for k in range(NV))
                    full = lax.fori_loop(0, L, row_step, full)
                    for k in range(NV):
                        sumbuf[pl.ds(k * L, L)] = full[1 + k]

            return last

        carry = jnp.int32(-1)
        for k in range(NV):
            sumbuf[pl.ds(k * L, L)] = zero

        start_fetch(0, 0)

        def chunk_body(j, carry):
            par = lax.rem(j, 2)

            @pl.when(par == 0)
            def _():
                wait_fetch(0)
                start_fetch(j + 1, 1)

            @pl.when(par == 1)
            def _():
                wait_fetch(1)
                start_fetch(j + 1, 0)

            s = base + j * CB
            win = win_of(s)
            g_lo = lax.div(s - win, L)
            g_hi = lax.div(jnp.minimum(s + CB, N_NODES) - win, L)
            roff0 = par * CB
            return lax.fori_loop(
                g_lo, g_hi,
                lambda g, cur: group_body(roff0, g, cur), carry)

        carry = lax.fori_loop(0, KMAX, chunk_body, carry)
        wait_fetch(0)

        cur = carry
        for k in range(NV):
            bbuf[pl.ds(D + k * L, L)] = sumbuf[pl.ds(k * L, L)]
        bidbuf[pl.ds(L, L)] = ones_i * cur

        pltpu.sync_copy(bbuf, bsum_hbm.at[wid])
        pltpu.sync_copy(bidbuf, bid_hbm.at[wid])

    return body(x1d, seg)


def _head(pd, bsums, bids, W, b2):
    def body(pd_ref, bs_ref, bi_ref, w_ref, b_ref, o_ref):
        y = pd_ref[0] + pd_ref[1]
        bs = bs_ref[...]
        bi = bi_ref[...]
        giota = lax.broadcasted_iota(jnp.int32, (G, NW), 0)
        for slot in range(2):
            bidv = bi[:, slot * L]
            oh = (giota == bidv[None, :]).astype(jnp.float32)
            y = y + jnp.dot(oh, bs[:, slot * D:(slot + 1) * D],
                            preferred_element_type=jnp.float32)
        y = jnp.maximum(y, 0.0)
        o_ref[...] = (
            jnp.dot(y, w_ref[...], preferred_element_type=jnp.float32)
            + b_ref[...]
        )

    return pl.pallas_call(
        body,
        out_shape=jax.ShapeDtypeStruct((G, NCLS), jnp.float32),
    )(pd, bsums, bids, W, b2)


def kernel(x, segment_ids, W, b):
    seg = segment_ids.astype(jnp.int32)
    pd, bsums, bids = _sc_segment_sum(x.reshape(-1), seg)
    return _head(pd.reshape(NC, G, D), bsums, bids, W, b.reshape(1, NCLS))


# speedup vs baseline: 2.2241x; 1.1460x over previous
import functools

import jax
import jax.numpy as jnp
from jax import lax
from jax.experimental import pallas as pl
from jax.experimental.pallas import tpu as pltpu
from jax.experimental.pallas import tpu_sc as plsc

N_NODES = 100000
D = 128
G = 512
NCLS = 10
L = 16
NC = 2
NS = 16
NW = NC * NS
CB = 400
KMAX = 6
PER_W = CB * KMAX
NSC = NW * PER_W
NV = D // L
TCB = 800
TGRID = (N_NODES - NSC) // TCB
assert NSC + TGRID * TCB == N_NODES and NSC % TCB == 0


def _sc_segment_sum(x1d, seg):
    mesh = plsc.VectorSubcoreMesh(core_axis_name="c", subcore_axis_name="s")

    @functools.partial(
        pl.kernel,
        out_type=(
            jax.ShapeDtypeStruct((NC, G * D), jnp.float32),
            jax.ShapeDtypeStruct((NW, 2 * D), jnp.float32),
            jax.ShapeDtypeStruct((NW, 2 * L), jnp.int32),
        ),
        mesh=mesh,
        scratch_types=[
            pltpu.VMEM((2 * CB * D,), jnp.float32),
            pltpu.VMEM((2 * CB + L,), jnp.int32),
            pltpu.VMEM((D,), jnp.float32),
            pltpu.VMEM((D,), jnp.float32),
            pltpu.VMEM((2 * D,), jnp.float32),
            pltpu.VMEM((2 * L,), jnp.int32),
            pltpu.VMEM((32 * D,), jnp.float32),
            pltpu.SMEM((1,), jnp.int32),
            pltpu.SemaphoreType.DMA,
            pltpu.SemaphoreType.DMA,
        ],
    )
    def body(x_hbm, ids_hbm, dense_hbm, bsum_hbm, bid_hbm,
             xbuf, idsbuf, stage, sumbuf, bbuf, bidbuf,
             zbuf, flagbuf, sem0, sem1):
        cid = lax.axis_index("c")
        sid = lax.axis_index("s")
        wid = sid * NC + cid
        zero = jnp.zeros((L,), jnp.float32)
        ones_i = jnp.full((L,), 1, jnp.int32)
        sems = (sem0, sem1)

        for i in range(32 * NV):
            zbuf[pl.ds(i * L, L)] = zero
        pltpu.sync_copy(zbuf, dense_hbm.at[cid, pl.ds(sid * 32 * D, 32 * D)])

        for k in range(NV):
            bbuf[pl.ds(k * L, L)] = zero
        bidbuf[pl.ds(0, L)] = ones_i * -1
        flagbuf[0] = 0

        plsc.subcore_barrier()

        base = wid * PER_W

        def win_of(s):
            return jnp.minimum(s, NSC - CB)

        def start_fetch(j, buf):
            win = win_of(base + j * CB)
            pltpu.async_copy(x_hbm.at[pl.ds(win * D, CB * D)],
                             xbuf.at[pl.ds(buf * (CB * D), CB * D)],
                             sems[buf])
            pltpu.async_copy(ids_hbm.at[pl.ds(win, CB)],
                             idsbuf.at[pl.ds(buf * CB, CB)], sems[buf])

        def wait_fetch(buf):
            pltpu.make_async_copy(x_hbm.at[pl.ds(0, CB * D)],
                                  xbuf.at[pl.ds(buf * (CB * D), CB * D)],
                                  sems[buf]).wait()
            pltpu.make_async_copy(ids_hbm.at[pl.ds(0, CB)],
                                  idsbuf.at[pl.ds(buf * CB, CB)],
                                  sems[buf]).wait()

        def tree_add(vals):
            vals = list(vals)
            while len(vals) > 1:
                nxt = [vals[i] + vals[i + 1] for i in range(0, len(vals) - 1, 2)]
                if len(vals) % 2:
                    nxt.append(vals[-1])
                vals = nxt
            return vals[0]

        def flush_to(seg_id, vals):
            f = flagbuf[0]

            @pl.when(f == 0)
            def _():
                flagbuf[0] = 1

            @pl.when(f == 1)
            def _():
                for k in range(NV):
                    bbuf[pl.ds(k * L, L)] = vals[k]
                bidbuf[pl.ds(0, L)] = ones_i * seg_id
                flagbuf[0] = 2

            @pl.when(f == 2)
            def _():
                for k in range(NV):
                    stage[pl.ds(k * L, L)] = vals[k]
                pltpu.sync_copy(stage, dense_hbm.at[cid, pl.ds(seg_id * D, D)])

        def group_body(roff0, g, cur):
            gro = roff0 + g * L
            idvec = idsbuf[pl.ds(gro, L)]
            gbase = gro * D
            first = idvec[0]
            last = idvec[L - 1]
            same = (first == last) & (first == cur)

            @pl.when(same)
            def _():
                for k in range(NV):
                    acc = tree_add(
                        [xbuf[pl.ds(gbase + r * D + k * L, L)]
                         for r in range(L)])
                    sumbuf[pl.ds(k * L, L)] = sumbuf[pl.ds(k * L, L)] + acc

            @pl.when(jnp.logical_not(same))
            def _():
                nchg = jnp.int32(0)
                for i in range(1, L):
                    nchg = nchg + (idvec[i] != idvec[i - 1]).astype(jnp.int32)
                mask_ok = jnp.where(first == cur, nchg == 1, first == last)

                @pl.when(mask_ok)
                def _():
                    conts = [idvec[r] == cur for r in range(L)]
                    flushvals = []
                    for k in range(NV):
                        rows = [xbuf[pl.ds(gbase + r * D + k * L, L)]
                                for r in range(L)]
                        total = tree_add(rows)
                        part = tree_add([jnp.where(conts[r], rows[r], zero)
                                         for r in range(L)])
                        flushvals.append(sumbuf[pl.ds(k * L, L)] + part)
                        sumbuf[pl.ds(k * L, L)] = total - part
                    flush_to(cur, flushvals)

                @pl.when(jnp.logical_not(mask_ok))
                def _():
                    def row_step(r, full):
                        cur_r, sums = full[0], full[1:]
                        rid = idsbuf[pl.ds(gro + r, L)][0]
                        changed = rid != cur_r

                        @pl.when(changed)
                        def _():
                            flush_to(cur_r, sums)

                        new_sums = tuple(
                            jnp.where(changed, zero, sums[k])
                            + xbuf[pl.ds(gbase + r * D + k * L, L)]
                            for k in range(NV)
                        )
                        return (rid,) + new_sums

                    full = (cur,) + tuple(sumbuf[pl.ds(k * L, L)]
                                          for k in range(NV))
                    full = lax.fori_loop(0, L, row_step, full)
                    for k in range(NV):
                        sumbuf[pl.ds(k * L, L)] = full[1 + k]

            return last

        carry = jnp.int32(-1)
        for k in range(NV):
            sumbuf[pl.ds(k * L, L)] = zero

        start_fetch(0, 0)

        def chunk_body(j, carry):
            par = lax.rem(j, 2)

            @pl.when(par == 0)
            def _():
                wait_fetch(0)
                start_fetch(j + 1, 1)

            @pl.when(par == 1)
            def _():
                wait_fetch(1)
                start_fetch(j + 1, 0)

            s = base + j * CB
            win = win_of(s)
            g_lo = lax.div(s - win, L)
            g_hi = lax.div(jnp.minimum(s + CB, NSC) - win, L)
            roff0 = par * CB
            return lax.fori_loop(
                g_lo, g_hi,
                lambda g, cur: group_body(roff0, g, cur), carry)

        carry = lax.fori_loop(0, KMAX, chunk_body, carry)
        wait_fetch(0)

        cur = carry
        for k in range(NV):
            bbuf[pl.ds(D + k * L, L)] = sumbuf[pl.ds(k * L, L)]
        bidbuf[pl.ds(L, L)] = ones_i * cur

        pltpu.sync_copy(bbuf, bsum_hbm.at[wid])
        pltpu.sync_copy(bidbuf, bid_hbm.at[wid])

    return body(x1d, seg)


def _tc_partial(x, seg3):
    def body(ids_ref, x_ref, o_ref):
        i = pl.program_id(0)
        ids_blk = ids_ref[0, 0, :]
        oh = (lax.broadcasted_iota(jnp.int32, (G, TCB), 0)
              == ids_blk[None, :]).astype(jnp.float32)
        p = jnp.dot(oh, x_ref[...], preferred_element_type=jnp.float32)

        @pl.when(i == 0)
        def _():
            o_ref[...] = p

        @pl.when(i > 0)
        def _():
            o_ref[...] = o_ref[...] + p

    base_blk = NSC // TCB
    return pl.pallas_call(
        body,
        grid=(TGRID,),
        in_specs=[
            pl.BlockSpec((1, 1, TCB), lambda i: (base_blk + i, 0, 0)),
            pl.BlockSpec((TCB, D), lambda i: (base_blk + i, 0)),
        ],
        out_specs=pl.BlockSpec((G, D), lambda i: (0, 0)),
        out_shape=jax.ShapeDtypeStruct((G, D), jnp.float32),
    )(seg3, x)


def _head(pd, tcp, bsums, bids, W, b2):
    def body(pd_ref, tc_ref, bs_ref, bi_ref, w_ref, b_ref, o_ref):
        y = pd_ref[0] + pd_ref[1] + tc_ref[...]
        bs = bs_ref[...]
        bi = bi_ref[...]
        giota = lax.broadcasted_iota(jnp.int32, (G, NW), 0)
        for slot in range(2):
            bidv = bi[:, slot * L]
            oh = (giota == bidv[None, :]).astype(jnp.float32)
            y = y + jnp.dot(oh, bs[:, slot * D:(slot + 1) * D],
                            preferred_element_type=jnp.float32)
        y = jnp.maximum(y, 0.0)
        o_ref[...] = (
            jnp.dot(y, w_ref[...], preferred_element_type=jnp.float32)
            + b_ref[...]
        )

    return pl.pallas_call(
        body,
        out_shape=jax.ShapeDtypeStruct((G, NCLS), jnp.float32),
    )(pd, tcp, bsums, bids, W, b2)


def kernel(x, segment_ids, W, b):
    seg = segment_ids.astype(jnp.int32)
    pd, bsums, bids = _sc_segment_sum(x.reshape(-1), seg)
    tcp = _tc_partial(x, seg.reshape(N_NODES // TCB, 1, TCB))
    return _head(pd.reshape(NC, G, D), tcp, bsums, bids, W,
                 b.reshape(1, NCLS))
